# baseline (speedup 1.0000x reference)
import functools

import jax
import jax.numpy as jnp
from jax import lax
from jax.experimental import pallas as pl
from jax.experimental.pallas import tpu as pltpu
from jax.experimental.pallas import tpu_sc as plsc

NN = 10000
EE = 320000
NC, NS = 2, 16
NW = NC * NS
C = 100
CS = 104
RPW = EE // NW // C
EPW = EE // NW
SL = 632
SLMAX = NN - SL

BN = 2000


def _make_agg(F):
    mesh = plsc.VectorSubcoreMesh(core_axis_name="c", subcore_axis_name="s",
                                  num_cores=NC, num_subcores=NS)

    @functools.partial(
        pl.kernel,
        mesh=mesh,
        out_type=(
            jax.ShapeDtypeStruct((NN, F), jnp.float32),
            jax.ShapeDtypeStruct((NN, F), jnp.float32),
        ),
        scratch_types=[
            pltpu.VMEM((RPW * CS,), jnp.int32),
            pltpu.VMEM((RPW, C), jnp.int32),
            pltpu.VMEM((C, F), jnp.float32),
            pltpu.VMEM((C, F), jnp.float32),
            pltpu.VMEM_SHARED((NN, F), jnp.float32),
            pltpu.SemaphoreType.DMA,
            pltpu.SemaphoreType.DMA,
        ],
    )
    def agg(g_hbm, z_hbm, src_hbm, dst_hbm, out0, out1, src_v, dst_v,
            rows0, rows1, acc_sh, sem0, sem1):
        c = lax.axis_index("c")
        s = lax.axis_index("s")
        row0 = jnp.minimum(s * SL, SLMAX)

        @pl.when(c == 0)
        def _():
            pltpu.sync_copy(g_hbm.at[pl.ds(row0, SL)],
                            acc_sh.at[pl.ds(row0, SL)])

        @pl.when(c == 1)
        def _():
            pltpu.sync_copy(z_hbm.at[pl.ds(row0, SL)],
                            acc_sh.at[pl.ds(row0, SL)])

        plsc.subcore_barrier()

        w = c * NS + s
        pltpu.sync_copy(src_hbm.at[w], src_v)
        pltpu.sync_copy(dst_hbm.at[w], dst_v)

        def sidx(j):
            return src_v.at[pl.ds(j * CS, C)]

        pltpu.async_copy(g_hbm.at[sidx(0)], rows0, sem0)

        def body(t, carry):
            j0 = 2 * t
            j1 = 2 * t + 1
            pltpu.make_async_copy(g_hbm.at[sidx(j0)], rows0, sem0).wait()
            pltpu.async_copy(g_hbm.at[sidx(j1)], rows1, sem1)
            pltpu.sync_copy(rows0, acc_sh.at[dst_v.at[j0]], add=True)

            @pl.when(j0 + 2 < RPW)
            def _():
                pltpu.async_copy(g_hbm.at[sidx(j0 + 2)], rows0, sem0)

            pltpu.make_async_copy(g_hbm.at[sidx(j1)], rows1, sem1).wait()
            pltpu.sync_copy(rows1, acc_sh.at[dst_v.at[j1]], add=True)
            return carry

        lax.fori_loop(0, RPW // 2, body, 0)

        if RPW % 2:
            pltpu.make_async_copy(g_hbm.at[sidx(RPW - 1)], rows0, sem0).wait()
            pltpu.sync_copy(rows0, acc_sh.at[dst_v.at[RPW - 1]], add=True)

        plsc.subcore_barrier()

        @pl.when(c == 0)
        def _():
            pltpu.sync_copy(acc_sh.at[pl.ds(row0, SL)],
                            out0.at[pl.ds(row0, SL)])

        @pl.when(c == 1)
        def _():
            pltpu.sync_copy(acc_sh.at[pl.ds(row0, SL)],
                            out1.at[pl.ds(row0, SL)])

    return agg


_get_agg = functools.lru_cache(maxsize=None)(_make_agg)


def _lrelu(x):
    return jnp.where(x > 0, x, 0.1 * x)


def _prep_body(p0_ref, p1_ref, x_ref, dis_ref, g1_ref):
    deg = p0_ref[:, :1] + p1_ref[:, :1]
    dis = lax.rsqrt(deg)
    dis_ref[...] = dis
    g1_ref[...] = x_ref[...] * dis


def _mid1_body(p0_ref, p1_ref, dis_ref, w1_ref, b1_ref, w2_ref, g2_ref):
    dis = dis_ref[...]
    u = (p0_ref[...] + p1_ref[...]) * dis
    h1 = _lrelu(jnp.dot(u, w1_ref[...], preferred_element_type=jnp.float32)
                + b1_ref[...])
    g2_ref[...] = jnp.dot(h1, w2_ref[...],
                          preferred_element_type=jnp.float32) * dis


def _mid_body(p0_ref, p1_ref, dis_ref, b_ref, w_ref, g_ref):
    dis = dis_ref[...]
    u = (p0_ref[...] + p1_ref[...]) * dis + b_ref[...]
    h = _lrelu(u)
    g_ref[...] = jnp.dot(h, w_ref[...],
                         preferred_element_type=jnp.float32) * dis


def _fin_body(p0_ref, p1_ref, dis_ref, b4_ref, wl_ref, bl_ref, out_ref):
    u = (p0_ref[...] + p1_ref[...]) * dis_ref[...] + b4_ref[...]
    h = _lrelu(u)[:, :32]
    logits = jnp.dot(h, wl_ref[...],
                     preferred_element_type=jnp.float32) + bl_ref[...]
    m = jnp.max(logits, axis=1, keepdims=True)
    e = jnp.exp(logits - m)
    out_ref[...] = e / jnp.sum(e, axis=1, keepdims=True)


def _row_spec(f):
    return pl.BlockSpec((BN, f), lambda i: (i, 0))


def _full_spec(shape):
    return pl.BlockSpec(shape, lambda i: tuple(0 for _ in shape))


_GRID = NN // BN


def _tc_call(body, in_specs, out_specs, out_shape, *args):
    return pl.pallas_call(
        body,
        grid=(_GRID,),
        in_specs=in_specs,
        out_specs=out_specs,
        out_shape=out_shape,
    )(*args)


def kernel(x, edge_index, W1, b1, W2, b2, W3, b3, W4, b4, Wl, bl):
    f32 = jnp.float32
    src2 = jnp.pad(edge_index[0].reshape(NW, RPW, C),
                   ((0, 0), (0, 0), (0, CS - C))).reshape(NW, RPW * CS)
    dst2 = edge_index[1].reshape(NW, RPW, C)

    ones128 = jnp.ones((NN, 128), f32)
    z128 = jnp.zeros((NN, 128), f32)

    W3p = jnp.pad(W3, ((0, 0), (0, 72)))
    b3p = jnp.pad(b3, (0, 72)).reshape(1, 128)
    W4p = jnp.pad(W4, ((0, 72), (0, 96)))
    b4p = jnp.pad(b4, (0, 96)).reshape(1, 128)
    b1r = b1.reshape(1, 256)
    b2r = b2.reshape(1, 128)
    blr = bl.reshape(1, 16)

    agg = _get_agg(128)

    src0 = jnp.zeros_like(src2)
    d0, d1 = agg(ones128, z128, src0, dst2)

    dis, g1 = _tc_call(
        _prep_body,
        [_row_spec(128), _row_spec(128), _row_spec(128)],
        (_row_spec(1), _row_spec(128)),
        (jax.ShapeDtypeStruct((NN, 1), f32),
         jax.ShapeDtypeStruct((NN, 128), f32)),
        d0, d1, x)

    a0, a1 = agg(g1, z128, src2, dst2)
    g2 = _tc_call(
        _mid1_body,
        [_row_spec(128), _row_spec(128), _row_spec(1),
         _full_spec((128, 256)), _full_spec((1, 256)), _full_spec((256, 128))],
        _row_spec(128),
        jax.ShapeDtypeStruct((NN, 128), f32),
        a0, a1, dis, W1, b1r, W2)

    a0, a1 = agg(g2, z128, src2, dst2)
    g3 = _tc_call(
        _mid_body,
        [_row_spec(128), _row_spec(128), _row_spec(1),
         _full_spec((1, 128)), _full_spec((128, 128))],
        _row_spec(128),
        jax.ShapeDtypeStruct((NN, 128), f32),
        a0, a1, dis, b2r, W3p)

    a0, a1 = agg(g3, z128, src2, dst2)
    g4 = _tc_call(
        _mid_body,
        [_row_spec(128), _row_spec(128), _row_spec(1),
         _full_spec((1, 128)), _full_spec((128, 128))],
        _row_spec(128),
        jax.ShapeDtypeStruct((NN, 128), f32),
        a0, a1, dis, b3p, W4p)

    a0, a1 = agg(g4, z128, src2, dst2)
    out = _tc_call(
        _fin_body,
        [_row_spec(128), _row_spec(128), _row_spec(1),
         _full_spec((1, 128)), _full_spec((32, 16)), _full_spec((1, 16))],
        _row_spec(16),
        jax.ShapeDtypeStruct((NN, 16), f32),
        a0, a1, dis, b4p, Wl, blr)

    return out


# speedup vs baseline: 19.0353x; 19.0353x over previous
import functools

import jax
import jax.numpy as jnp
from jax import lax
from jax.experimental import pallas as pl
from jax.experimental.pallas import tpu as pltpu
from jax.experimental.pallas import tpu_sc as plsc

NN = 10000
EE = 320000
NC, NS = 2, 16
NW = NC * NS
C = 100
CS = 104
RPW = EE // NW // C
EPW = EE // NW
SL = 632
SLMAX = NN - SL

BN = 2000


def _make_agg(F):
    mesh = plsc.VectorSubcoreMesh(core_axis_name="c", subcore_axis_name="s",
                                  num_cores=NC, num_subcores=NS)

    @functools.partial(
        pl.kernel,
        mesh=mesh,
        out_type=(
            jax.ShapeDtypeStruct((NN, F), jnp.float32),
            jax.ShapeDtypeStruct((NN, F), jnp.float32),
        ),
        scratch_types=[
            pltpu.VMEM((RPW * CS,), jnp.int32),
            pltpu.VMEM((RPW, C), jnp.int32),
            pltpu.VMEM((C, F), jnp.float32),
            pltpu.VMEM((C, F), jnp.float32),
            pltpu.VMEM_SHARED((NN, F), jnp.float32),
            pltpu.SemaphoreType.DMA,
            pltpu.SemaphoreType.DMA,
        ],
    )
    def agg(g_hbm, z_hbm, src_hbm, dst_hbm, out0, out1, src_v, dst_v,
            rows0, rows1, acc_sh, sem0, sem1):
        c = lax.axis_index("c")
        s = lax.axis_index("s")
        row0 = jnp.minimum(s * SL, SLMAX)

        @pl.when(c == 0)
        def _():
            pltpu.sync_copy(g_hbm.at[pl.ds(row0, SL)],
                            acc_sh.at[pl.ds(row0, SL)])

        @pl.when(c == 1)
        def _():
            pltpu.sync_copy(z_hbm.at[pl.ds(row0, SL)],
                            acc_sh.at[pl.ds(row0, SL)])

        plsc.subcore_barrier()

        w = c * NS + s
        pltpu.sync_copy(src_hbm.at[w], src_v)
        pltpu.sync_copy(dst_hbm.at[w], dst_v)

        def sidx(j):
            return src_v.at[pl.ds(j * CS, C)]

        pltpu.async_copy(g_hbm.at[sidx(0)], rows0, sem0)

        def body(t, carry):
            j0 = 2 * t
            j1 = 2 * t + 1
            pltpu.make_async_copy(g_hbm.at[sidx(j0)], rows0, sem0).wait()
            pltpu.async_copy(g_hbm.at[sidx(j1)], rows1, sem1)
            pltpu.sync_copy(rows0, acc_sh.at[dst_v.at[j0]], add=True)

            @pl.when(j0 + 2 < RPW)
            def _():
                pltpu.async_copy(g_hbm.at[sidx(j0 + 2)], rows0, sem0)

            pltpu.make_async_copy(g_hbm.at[sidx(j1)], rows1, sem1).wait()
            pltpu.sync_copy(rows1, acc_sh.at[dst_v.at[j1]], add=True)
            return carry

        lax.fori_loop(0, RPW // 2, body, 0)

        if RPW % 2:
            pltpu.make_async_copy(g_hbm.at[sidx(RPW - 1)], rows0, sem0).wait()
            pltpu.sync_copy(rows0, acc_sh.at[dst_v.at[RPW - 1]], add=True)

        plsc.subcore_barrier()

        @pl.when(c == 0)
        def _():
            pltpu.sync_copy(acc_sh.at[pl.ds(row0, SL)],
                            out0.at[pl.ds(row0, SL)])

        @pl.when(c == 1)
        def _():
            pltpu.sync_copy(acc_sh.at[pl.ds(row0, SL)],
                            out1.at[pl.ds(row0, SL)])

    return agg


_get_agg = functools.lru_cache(maxsize=None)(_make_agg)


def _lrelu(x):
    return jnp.where(x > 0, x, 0.1 * x)


def _prep_body(p0_ref, p1_ref, x_ref, dis_ref, g1_ref):
    deg = p0_ref[:, :1] + p1_ref[:, :1]
    dis = lax.rsqrt(deg)
    dis_ref[...] = dis
    g1_ref[...] = x_ref[...] * dis


def _mid1_body(p0_ref, p1_ref, dis_ref, w1_ref, b1_ref, w2_ref, g2_ref):
    dis = dis_ref[...]
    u = (p0_ref[...] + p1_ref[...]) * dis
    h1 = _lrelu(jnp.dot(u, w1_ref[...], preferred_element_type=jnp.float32)
                + b1_ref[...])
    g2_ref[...] = jnp.dot(h1, w2_ref[...],
                          preferred_element_type=jnp.float32) * dis


def _mid_body(p0_ref, p1_ref, dis_ref, b_ref, w_ref, g_ref):
    dis = dis_ref[...]
    u = (p0_ref[...] + p1_ref[...]) * dis + b_ref[...]
    h = _lrelu(u)
    g_ref[...] = jnp.dot(h, w_ref[...],
                         preferred_element_type=jnp.float32) * dis


def _fin_body(p0_ref, p1_ref, dis_ref, b4_ref, wl_ref, bl_ref, out_ref):
    u = (p0_ref[...] + p1_ref[...]) * dis_ref[...] + b4_ref[...]
    h = _lrelu(u)[:, :32]
    logits = jnp.dot(h, wl_ref[...],
                     preferred_element_type=jnp.float32) + bl_ref[...]
    m = jnp.max(logits, axis=1, keepdims=True)
    e = jnp.exp(logits - m)
    out_ref[...] = e / jnp.sum(e, axis=1, keepdims=True)


def _row_spec(f):
    return pl.BlockSpec((BN, f), lambda i: (i, 0))


def _full_spec(shape):
    return pl.BlockSpec(shape, lambda i: tuple(0 for _ in shape))


_GRID = NN // BN


def _tc_call(body, in_specs, out_specs, out_shape, *args):
    return pl.pallas_call(
        body,
        grid=(_GRID,),
        in_specs=in_specs,
        out_specs=out_specs,
        out_shape=out_shape,
    )(*args)


def kernel(x, edge_index, W1, b1, W2, b2, W3, b3, W4, b4, Wl, bl):
    f32 = jnp.float32
    src2 = jnp.pad(edge_index[0].reshape(NW, RPW, C),
                   ((0, 0), (0, 0), (0, CS - C))).reshape(NW, RPW * CS)
    dst2 = edge_index[1].reshape(NW, RPW, C)

    ones128 = jnp.ones((NN, 128), f32)
    z128 = jnp.zeros((NN, 128), f32)

    W3p = jnp.pad(W3, ((0, 0), (0, 72)))
    b3p = jnp.pad(b3, (0, 72)).reshape(1, 128)
    W4p = jnp.pad(W4, ((0, 72), (0, 96)))
    b4p = jnp.pad(b4, (0, 96)).reshape(1, 128)
    b1r = b1.reshape(1, 256)
    b2r = b2.reshape(1, 128)
    blr = bl.reshape(1, 16)

    agg = _get_agg(128)

    d0, d1 = agg(ones128, z128, src2, dst2)

    dis, g1 = _tc_call(
        _prep_body,
        [_row_spec(128), _row_spec(128), _row_spec(128)],
        (_row_spec(1), _row_spec(128)),
        (jax.ShapeDtypeStruct((NN, 1), f32),
         jax.ShapeDtypeStruct((NN, 128), f32)),
        d0, d1, x)

    a0, a1 = agg(g1, z128, src2, dst2)
    g2 = _tc_call(
        _mid1_body,
        [_row_spec(128), _row_spec(128), _row_spec(1),
         _full_spec((128, 256)), _full_spec((1, 256)), _full_spec((256, 128))],
        _row_spec(128),
        jax.ShapeDtypeStruct((NN, 128), f32),
        a0, a1, dis, W1, b1r, W2)

    a0, a1 = agg(g2, z128, src2, dst2)
    g3 = _tc_call(
        _mid_body,
        [_row_spec(128), _row_spec(128), _row_spec(1),
         _full_spec((1, 128)), _full_spec((128, 128))],
        _row_spec(128),
        jax.ShapeDtypeStruct((NN, 128), f32),
        a0, a1, dis, b2r, W3p)

    a0, a1 = agg(g3, z128, src2, dst2)
    g4 = _tc_call(
        _mid_body,
        [_row_spec(128), _row_spec(128), _row_spec(1),
         _full_spec((1, 128)), _full_spec((128, 128))],
        _row_spec(128),
        jax.ShapeDtypeStruct((NN, 128), f32),
        a0, a1, dis, b3p, W4p)

    a0, a1 = agg(g4, z128, src2, dst2)
    out = _tc_call(
        _fin_body,
        [_row_spec(128), _row_spec(128), _row_spec(1),
         _full_spec((1, 128)), _full_spec((32, 16)), _full_spec((1, 16))],
        _row_spec(16),
        jax.ShapeDtypeStruct((NN, 16), f32),
        a0, a1, dis, b4p, Wl, blr)

    return out
